# baseline (speedup 1.0000x reference)
import functools

import jax
import jax.numpy as jnp
from jax import lax
from jax.experimental import pallas as pl
from jax.experimental.pallas import tpu as pltpu
from jax.experimental.pallas import tpu_sc as plsc

D_MODEL = 64

NC = 2
NS = 16
NW = NC * NS

GS = (96, 104)
RW = 2
NB = 2


@jax.jit
def _gather_rows(weight, x):
  n_rows, seq = x.shape
  r_per_w = n_rows // NW
  n_outer = r_per_w // (RW * NB)

  mesh = plsc.VectorSubcoreMesh(
      core_axis_name="c", subcore_axis_name="s", num_cores=NC, num_subcores=NS
  )

  @functools.partial(
      pl.kernel,
      mesh=mesh,
      compiler_params=pltpu.CompilerParams(use_tc_tiling_on_sc=False),
      out_type=jax.ShapeDtypeStruct((n_rows, seq, D_MODEL), jnp.float32),
      scratch_types=(
          [pltpu.VMEM((r_per_w, seq), jnp.int32),
           pltpu.VMEM((NB, RW, seq, D_MODEL), jnp.float32)]
          + [pltpu.SemaphoreType.DMA] * (NB * RW * 2 + NB)
      ),
  )
  def k(table_hbm, idx_hbm, out_hbm, idx_v, rows_v, *sems):
    gsem = sems[:NB * RW * 2]
    osem = sems[NB * RW * 2:]
    wid = lax.axis_index("s") * NC + lax.axis_index("c")
    base = wid * r_per_w
    pltpu.sync_copy(idx_hbm.at[pl.ds(base, r_per_w)], idx_v)

    def outer(i, carry):
      r0 = i * NB * RW
      for o in range(NB):
        @pl.when(i > 0)
        def _wait_out():
          pltpu.make_async_copy(
              rows_v.at[o],
              out_hbm.at[pl.ds(base + r0 + o * RW - NB * RW, RW)],
              osem[o],
          ).wait()

        for j in range(RW):
          r = r0 + o * RW + j
          for h, (off, g) in enumerate(zip((0, GS[0]), GS)):
            pltpu.async_copy(
                table_hbm.at[idx_v.at[r, pl.ds(off, g)]],
                rows_v.at[o, j, pl.ds(off, g)],
                gsem[(o * RW + j) * 2 + h])
      for o in range(NB):
        for j in range(RW):
          r = r0 + o * RW + j
          for h, (off, g) in enumerate(zip((0, GS[0]), GS)):
            pltpu.make_async_copy(
                table_hbm.at[idx_v.at[r, pl.ds(off, g)]],
                rows_v.at[o, j, pl.ds(off, g)],
                gsem[(o * RW + j) * 2 + h]).wait()
        pltpu.async_copy(rows_v.at[o],
                         out_hbm.at[pl.ds(base + r0 + o * RW, RW)],
                         osem[o])
      return carry

    lax.fori_loop(0, n_outer, outer, 0)
    for o in range(NB):
      pltpu.make_async_copy(
          rows_v.at[o],
          out_hbm.at[pl.ds(base + (n_outer - 1) * NB * RW + o * RW, RW)],
          osem[o],
      ).wait()

  return k(weight, x)


def kernel(x, weight):
  return _gather_rows(weight, x.astype(jnp.int32))


# speedup vs baseline: 1.3311x; 1.3311x over previous
import functools

import jax
import jax.numpy as jnp
from jax import lax
from jax.experimental import pallas as pl
from jax.experimental.pallas import tpu as pltpu
from jax.experimental.pallas import tpu_sc as plsc

D_MODEL = 64
D_PAD = 128

NC = 2
NS = 16
NW = NC * NS

GS = (96, 104)
RW = 2
NB = 2


@jax.jit
def _gather_rows(weight, x):
  n_rows, seq = x.shape
  r_per_w = n_rows // NW
  n_outer = r_per_w // (RW * NB)

  mesh = plsc.VectorSubcoreMesh(
      core_axis_name="c", subcore_axis_name="s", num_cores=NC, num_subcores=NS
  )

  @functools.partial(
      pl.kernel,
      mesh=mesh,
      compiler_params=pltpu.CompilerParams(use_tc_tiling_on_sc=False),
      out_type=jax.ShapeDtypeStruct((n_rows * seq, D_PAD), jnp.float32),
      scratch_types=(
          [pltpu.VMEM((r_per_w, seq), jnp.int32),
           pltpu.VMEM((NB, RW * seq, D_MODEL), jnp.float32)]
          + [pltpu.SemaphoreType.DMA] * (NB * RW * 2 + NB)
      ),
  )
  def k(table_hbm, idx_hbm, out_hbm, idx_v, rows_v, *sems):
    gsem = sems[:NB * RW * 2]
    osem = sems[NB * RW * 2:]
    wid = lax.axis_index("s") * NC + lax.axis_index("c")
    base = wid * r_per_w
    pltpu.sync_copy(idx_hbm.at[pl.ds(base, r_per_w)], idx_v)

    def outer(i, carry):
      r0 = i * NB * RW
      for o in range(NB):
        @pl.when(i > 0)
        def _wait_out():
          pltpu.make_async_copy(
              rows_v.at[o],
              out_hbm.at[pl.ds((base + r0 + o * RW - NB * RW) * seq,
                               RW * seq),
                         pl.ds(0, D_MODEL)],
              osem[o],
          ).wait()

        for j in range(RW):
          r = r0 + o * RW + j
          for h, (off, g) in enumerate(zip((0, GS[0]), GS)):
            pltpu.async_copy(
                table_hbm.at[idx_v.at[r, pl.ds(off, g)]],
                rows_v.at[o, pl.ds(j * seq + off, g)],
                gsem[(o * RW + j) * 2 + h])
      for o in range(NB):
        for j in range(RW):
          r = r0 + o * RW + j
          for h, (off, g) in enumerate(zip((0, GS[0]), GS)):
            pltpu.make_async_copy(
                table_hbm.at[idx_v.at[r, pl.ds(off, g)]],
                rows_v.at[o, pl.ds(j * seq + off, g)],
                gsem[(o * RW + j) * 2 + h]).wait()
        pltpu.async_copy(
            rows_v.at[o],
            out_hbm.at[pl.ds((base + r0 + o * RW) * seq, RW * seq),
                       pl.ds(0, D_MODEL)],
            osem[o])
      return carry

    lax.fori_loop(0, n_outer, outer, 0)
    for o in range(NB):
      pltpu.make_async_copy(
          rows_v.at[o],
          out_hbm.at[pl.ds((base + (n_outer - 1) * NB * RW + o * RW) * seq,
                           RW * seq),
                     pl.ds(0, D_MODEL)],
          osem[o],
      ).wait()

  return k(weight, x)


def kernel(x, weight):
  n_rows, seq = x.shape
  out128 = _gather_rows(weight, x.astype(jnp.int32))
  return out128[:, :D_MODEL].reshape(n_rows, seq, D_MODEL)
